# baseline (speedup 1.0000x reference)
import functools

import jax
import jax.numpy as jnp
from jax import lax
from jax.experimental import pallas as pl
from jax.experimental.pallas import tpu as pltpu
from jax.experimental.pallas import tpu_sc as plsc

B, T = 1024, 50
N_NUM, N_EMB = 13, 26
EMB_DIM = 32
VOCAB = 100000
F = N_NUM + N_EMB * EMB_DIM
FE = N_EMB * EMB_DIM
EPS = 1e-5
BT = B * T

NC, NS = 2, 16
NW = NC * NS
RPW = FE // NW
TCK = 6400
NTC = BT // TCK


def _sc_gather(tables_2d, idx_tm):
    mesh = plsc.VectorSubcoreMesh(core_axis_name="c", subcore_axis_name="s")

    @functools.partial(
        pl.kernel,
        mesh=mesh,
        out_type=jax.ShapeDtypeStruct((FE, BT), jnp.float32),
        scratch_types=[
            pltpu.VMEM((VOCAB,), jnp.float32),
            pltpu.VMEM((TCK,), jnp.int32),
            pltpu.VMEM((TCK,), jnp.float32),
        ],
        compiler_params=pltpu.CompilerParams(use_tc_tiling_on_sc=False,
                                             needs_layout_passes=False),
    )
    def gather_k(tbl_hbm, idx_hbm, out_hbm, row_v, idx_v, out_v):
        wid = lax.axis_index("s") * NC + lax.axis_index("c")

        def rbody(rr, _):
            rf = wid * RPW + rr
            j = rf // EMB_DIM
            pltpu.sync_copy(tbl_hbm.at[rf], row_v)

            def cbody(c, _):
                pltpu.sync_copy(idx_hbm.at[j, pl.ds(c * TCK, TCK)], idx_v)

                def gbody(g, _):
                    base = g * 128
                    for u in range(8):
                        iv = idx_v[pl.ds(base + u * 16, 16)]
                        out_v[pl.ds(base + u * 16, 16)] = (
                            plsc.load_gather(row_v, [iv]))
                    return 0

                lax.fori_loop(0, TCK // 128, gbody, 0)
                pltpu.sync_copy(out_v, out_hbm.at[rf, pl.ds(c * TCK, TCK)])
                return 0

            lax.fori_loop(0, NTC, cbody, 0)
            return 0

        lax.fori_loop(0, RPW, rbody, 0)

    return gather_k(tables_2d, idx_tm)


CK = 2048
NBC = BT // CK


def _stats_body(len_ref, num_ref, mask_ref, emb_ref,
                gn_ref, ge_ref, bn_ref, be_ref,
                sn_ref, se_ref, cn_ref, ce_ref,
                acc_sn, acc_qn, acc_se, acc_qe, acc_n):
    i = pl.program_id(0)

    @pl.when(i == 0)
    def _():
        lens = len_ref[...]
        lf = lens.astype(jnp.float32)
        acc_n[...] = jnp.sum(lf).reshape(1, 1)
        m3 = (lax.broadcasted_iota(jnp.int32, (T, 1, B), 0)
              < lens.reshape(1, 1, B)).astype(jnp.float32)
        num = num_ref[...]
        nm = num * m3
        acc_sn[...] = jnp.sum(nm, axis=(0, 2)).reshape(N_NUM, 1)
        acc_qn[...] = jnp.sum(nm * num, axis=(0, 2)).reshape(N_NUM, 1)
        acc_se[...] = jnp.zeros_like(acc_se)
        acc_qe[...] = jnp.zeros_like(acc_qe)

    emb = emb_ref[...]
    em = emb * mask_ref[...]
    acc_se[...] += jnp.sum(em, axis=1).reshape(FE, 1)
    acc_qe[...] += jnp.sum(em * emb, axis=1).reshape(FE, 1)

    @pl.when(i == pl.num_programs(0) - 1)
    def _():
        inv_n = 1.0 / acc_n[0, 0]
        mean_n = acc_sn[...] * inv_n
        var_n = jnp.maximum(acc_qn[...] * inv_n - mean_n * mean_n, 0.0)
        s_n = gn_ref[...] * lax.rsqrt(var_n + EPS)
        sn_ref[...] = s_n
        cn_ref[...] = bn_ref[...] - mean_n * s_n
        mean_e = acc_se[...] * inv_n
        var_e = jnp.maximum(acc_qe[...] * inv_n - mean_e * mean_e, 0.0)
        s_e = ge_ref[...] * lax.rsqrt(var_e + EPS)
        se_ref[...] = s_e
        ce_ref[...] = be_ref[...] - mean_e * s_e


def _stats(len_row, numeric_t, mask_tm, emb_fm,
           gamma_n, gamma_e, beta_n, beta_e):
    return pl.pallas_call(
        _stats_body,
        grid=(NBC,),
        in_specs=[
            pl.BlockSpec((1, B), lambda i: (0, 0)),
            pl.BlockSpec((T, N_NUM, B), lambda i: (0, 0, 0)),
            pl.BlockSpec((1, CK), lambda i: (0, i)),
            pl.BlockSpec((FE, CK), lambda i: (0, i)),
            pl.BlockSpec((N_NUM, 1), lambda i: (0, 0)),
            pl.BlockSpec((FE, 1), lambda i: (0, 0)),
            pl.BlockSpec((N_NUM, 1), lambda i: (0, 0)),
            pl.BlockSpec((FE, 1), lambda i: (0, 0)),
        ],
        out_specs=[
            pl.BlockSpec((N_NUM, 1), lambda i: (0, 0)),
            pl.BlockSpec((FE, 1), lambda i: (0, 0)),
            pl.BlockSpec((N_NUM, 1), lambda i: (0, 0)),
            pl.BlockSpec((FE, 1), lambda i: (0, 0)),
        ],
        out_shape=[
            jax.ShapeDtypeStruct((N_NUM, 1), jnp.float32),
            jax.ShapeDtypeStruct((FE, 1), jnp.float32),
            jax.ShapeDtypeStruct((N_NUM, 1), jnp.float32),
            jax.ShapeDtypeStruct((FE, 1), jnp.float32),
        ],
        scratch_shapes=[
            pltpu.VMEM((N_NUM, 1), jnp.float32),
            pltpu.VMEM((N_NUM, 1), jnp.float32),
            pltpu.VMEM((FE, 1), jnp.float32),
            pltpu.VMEM((FE, 1), jnp.float32),
            pltpu.VMEM((1, 1), jnp.float32),
        ],
    )(len_row, numeric_t, mask_tm, emb_fm, gamma_n, gamma_e, beta_n, beta_e)


TB = 1


def _norm_body(len_ref, num_ref, emb_ref, sn_ref, se_ref, cn_ref, ce_ref,
               out_ref):
    it = pl.program_id(0)
    lens = len_ref[...]
    s_n, c_n = sn_ref[...], cn_ref[...]
    s_e, c_e = se_ref[...], ce_ref[...]
    for k in range(TB):
        t = it * TB + k
        m = (t < lens).astype(jnp.float32)
        e = emb_ref[:, k * B:(k + 1) * B]
        out_ref[k, N_NUM:F, :] = (e * s_e + c_e) * m
        nt = num_ref[k]
        out_ref[k, 0:N_NUM, :] = (nt * s_n + c_n) * m


def _norm(len_row, numeric_t, emb_fm, s_n, s_e, c_n, c_e):
    out_t = pl.pallas_call(
        _norm_body,
        grid=(T // TB,),
        in_specs=[
            pl.BlockSpec((1, B), lambda it: (0, 0)),
            pl.BlockSpec((TB, N_NUM, B), lambda it: (it, 0, 0)),
            pl.BlockSpec((FE, TB * B), lambda it: (0, it)),
            pl.BlockSpec((N_NUM, 1), lambda it: (0, 0)),
            pl.BlockSpec((FE, 1), lambda it: (0, 0)),
            pl.BlockSpec((N_NUM, 1), lambda it: (0, 0)),
            pl.BlockSpec((FE, 1), lambda it: (0, 0)),
        ],
        out_specs=pl.BlockSpec((TB, F, B), lambda it: (it, 0, 0)),
        out_shape=jax.ShapeDtypeStruct((T, F, B), jnp.float32),
        compiler_params=pltpu.CompilerParams(
            vmem_limit_bytes=100 * 1024 * 1024),
    )(len_row, numeric_t, emb_fm, s_n, s_e, c_n, c_e)
    return jnp.transpose(out_t, (2, 0, 1))


def kernel(numeric, emb_idx, lengths, tables, gamma, beta):
    tables_2d = jnp.transpose(tables, (0, 2, 1)).reshape(FE, VOCAB)
    idx_tm = jnp.transpose(emb_idx, (2, 1, 0)).reshape(N_EMB, BT)
    idx_tm = idx_tm.astype(jnp.int32)
    numeric_t = jnp.transpose(numeric, (1, 2, 0))
    len_row = lengths.reshape(1, B).astype(jnp.int32)
    mask_tm = (jnp.arange(T, dtype=jnp.int32)[:, None]
               < lengths[None, :]).astype(jnp.float32).reshape(1, BT)
    emb_fm = _sc_gather(tables_2d, idx_tm)
    gamma_n = gamma[:N_NUM].reshape(N_NUM, 1)
    gamma_e = gamma[N_NUM:].reshape(FE, 1)
    beta_n = beta[:N_NUM].reshape(N_NUM, 1)
    beta_e = beta[N_NUM:].reshape(FE, 1)
    s_n, s_e, c_n, c_e = _stats(len_row, numeric_t, mask_tm, emb_fm,
                                gamma_n, gamma_e, beta_n, beta_e)
    return _norm(len_row, numeric_t, emb_fm, s_n, s_e, c_n, c_e)


# speedup vs baseline: 1.1752x; 1.1752x over previous
import functools

import jax
import jax.numpy as jnp
from jax import lax
from jax.experimental import pallas as pl
from jax.experimental.pallas import tpu as pltpu
from jax.experimental.pallas import tpu_sc as plsc

B, T = 1024, 50
N_NUM, N_EMB = 13, 26
EMB_DIM = 32
VOCAB = 100000
F = N_NUM + N_EMB * EMB_DIM
FE = N_EMB * EMB_DIM
EPS = 1e-5
BT = B * T

NC, NS = 2, 16
NW = NC * NS
RPW = FE // NW
TCK = 6400
NTC = BT // TCK


def _sc_gather(tables_2d, idx_tm):
    mesh = plsc.VectorSubcoreMesh(core_axis_name="c", subcore_axis_name="s")

    @functools.partial(
        pl.kernel,
        mesh=mesh,
        out_type=jax.ShapeDtypeStruct((FE, BT), jnp.float32),
        scratch_types=[
            pltpu.VMEM((VOCAB,), jnp.float32),
            pltpu.VMEM((2, TCK), jnp.int32),
            pltpu.VMEM((2, TCK), jnp.float32),
            pltpu.SemaphoreType.DMA,
            pltpu.SemaphoreType.DMA,
            pltpu.SemaphoreType.DMA,
            pltpu.SemaphoreType.DMA,
        ],
        compiler_params=pltpu.CompilerParams(use_tc_tiling_on_sc=False,
                                             needs_layout_passes=False),
    )
    def gather_k(tbl_hbm, idx_hbm, out_hbm, row_v, idx_v, out_v,
                 isem_a, isem_b, osem_a, osem_b):
        wid = lax.axis_index("s") * NC + lax.axis_index("c")
        isems = (isem_a, isem_b)
        osems = (osem_a, osem_b)

        def idx_start(j, c, b):
            pltpu.async_copy(idx_hbm.at[j, pl.ds(c * TCK, TCK)],
                             idx_v.at[b], isems[b])

        def idx_wait(j, c, b):
            pltpu.make_async_copy(idx_hbm.at[j, pl.ds(c * TCK, TCK)],
                                  idx_v.at[b], isems[b]).wait()

        def out_start(rf, c, b):
            pltpu.async_copy(out_v.at[b],
                             out_hbm.at[rf, pl.ds(c * TCK, TCK)], osems[b])

        def out_wait(rf, c, b):
            pltpu.make_async_copy(
                out_v.at[b], out_hbm.at[rf, pl.ds(c * TCK, TCK)],
                osems[b]).wait()

        def compute(b):
            def gbody(g, _):
                base = g * 128
                for u in range(8):
                    iv = idx_v[b, pl.ds(base + u * 16, 16)]
                    out_v[b, pl.ds(base + u * 16, 16)] = (
                        plsc.load_gather(row_v, [iv]))
                return 0

            lax.fori_loop(0, TCK // 128, gbody, 0)

        def rbody(rr, _):
            rf = wid * RPW + rr
            j = rf // EMB_DIM
            pltpu.sync_copy(tbl_hbm.at[rf], row_v)
            idx_start(j, 0, 0)
            idx_start(j, 1, 1)

            def pbody(p, _):
                for b in range(2):
                    c = p * 2 + b
                    idx_wait(j, c, b)

                    @pl.when(c >= 2)
                    def _():
                        out_wait(rf, c - 2, b)

                    compute(b)

                    @pl.when(c + 2 < NTC)
                    def _():
                        idx_start(j, c + 2, b)

                    out_start(rf, c, b)
                return 0

            lax.fori_loop(0, NTC // 2, pbody, 0)
            out_wait(rf, NTC - 2, 0)
            out_wait(rf, NTC - 1, 1)
            return 0

        lax.fori_loop(0, RPW, rbody, 0)

    return gather_k(tables_2d, idx_tm)


CK = 2048
NBC = BT // CK


def _stats_body(len_ref, num_ref, mask_ref, emb_ref,
                gn_ref, ge_ref, bn_ref, be_ref,
                sn_ref, se_ref, cn_ref, ce_ref,
                acc_sn, acc_qn, acc_se, acc_qe, acc_n):
    i = pl.program_id(0)

    @pl.when(i == 0)
    def _():
        lens = len_ref[...]
        lf = lens.astype(jnp.float32)
        acc_n[...] = jnp.sum(lf).reshape(1, 1)
        m3 = (lax.broadcasted_iota(jnp.int32, (T, 1, B), 0)
              < lens.reshape(1, 1, B)).astype(jnp.float32)
        num = num_ref[...]
        nm = num * m3
        acc_sn[...] = jnp.sum(nm, axis=(0, 2)).reshape(N_NUM, 1)
        acc_qn[...] = jnp.sum(nm * num, axis=(0, 2)).reshape(N_NUM, 1)
        acc_se[...] = jnp.zeros_like(acc_se)
        acc_qe[...] = jnp.zeros_like(acc_qe)

    emb = emb_ref[...]
    em = emb * mask_ref[...]
    acc_se[...] += jnp.sum(em, axis=1).reshape(FE, 1)
    acc_qe[...] += jnp.sum(em * emb, axis=1).reshape(FE, 1)

    @pl.when(i == pl.num_programs(0) - 1)
    def _():
        inv_n = 1.0 / acc_n[0, 0]
        mean_n = acc_sn[...] * inv_n
        var_n = jnp.maximum(acc_qn[...] * inv_n - mean_n * mean_n, 0.0)
        s_n = gn_ref[...] * lax.rsqrt(var_n + EPS)
        sn_ref[...] = s_n
        cn_ref[...] = bn_ref[...] - mean_n * s_n
        mean_e = acc_se[...] * inv_n
        var_e = jnp.maximum(acc_qe[...] * inv_n - mean_e * mean_e, 0.0)
        s_e = ge_ref[...] * lax.rsqrt(var_e + EPS)
        se_ref[...] = s_e
        ce_ref[...] = be_ref[...] - mean_e * s_e


def _stats(len_row, numeric_t, mask_tm, emb_fm,
           gamma_n, gamma_e, beta_n, beta_e):
    return pl.pallas_call(
        _stats_body,
        grid=(NBC,),
        in_specs=[
            pl.BlockSpec((1, B), lambda i: (0, 0)),
            pl.BlockSpec((T, N_NUM, B), lambda i: (0, 0, 0)),
            pl.BlockSpec((1, CK), lambda i: (0, i)),
            pl.BlockSpec((FE, CK), lambda i: (0, i)),
            pl.BlockSpec((N_NUM, 1), lambda i: (0, 0)),
            pl.BlockSpec((FE, 1), lambda i: (0, 0)),
            pl.BlockSpec((N_NUM, 1), lambda i: (0, 0)),
            pl.BlockSpec((FE, 1), lambda i: (0, 0)),
        ],
        out_specs=[
            pl.BlockSpec((N_NUM, 1), lambda i: (0, 0)),
            pl.BlockSpec((FE, 1), lambda i: (0, 0)),
            pl.BlockSpec((N_NUM, 1), lambda i: (0, 0)),
            pl.BlockSpec((FE, 1), lambda i: (0, 0)),
        ],
        out_shape=[
            jax.ShapeDtypeStruct((N_NUM, 1), jnp.float32),
            jax.ShapeDtypeStruct((FE, 1), jnp.float32),
            jax.ShapeDtypeStruct((N_NUM, 1), jnp.float32),
            jax.ShapeDtypeStruct((FE, 1), jnp.float32),
        ],
        scratch_shapes=[
            pltpu.VMEM((N_NUM, 1), jnp.float32),
            pltpu.VMEM((N_NUM, 1), jnp.float32),
            pltpu.VMEM((FE, 1), jnp.float32),
            pltpu.VMEM((FE, 1), jnp.float32),
            pltpu.VMEM((1, 1), jnp.float32),
        ],
    )(len_row, numeric_t, mask_tm, emb_fm, gamma_n, gamma_e, beta_n, beta_e)


TB = 1


def _norm_body(len_ref, num_ref, emb_ref, sn_ref, se_ref, cn_ref, ce_ref,
               out_ref):
    it = pl.program_id(0)
    lens = len_ref[...]
    s_n, c_n = sn_ref[...], cn_ref[...]
    s_e, c_e = se_ref[...], ce_ref[...]
    for k in range(TB):
        t = it * TB + k
        m = (t < lens).astype(jnp.float32)
        e = emb_ref[:, k * B:(k + 1) * B]
        out_ref[k, N_NUM:F, :] = (e * s_e + c_e) * m
        nt = num_ref[k]
        out_ref[k, 0:N_NUM, :] = (nt * s_n + c_n) * m


def _norm(len_row, numeric_t, emb_fm, s_n, s_e, c_n, c_e):
    out_t = pl.pallas_call(
        _norm_body,
        grid=(T // TB,),
        in_specs=[
            pl.BlockSpec((1, B), lambda it: (0, 0)),
            pl.BlockSpec((TB, N_NUM, B), lambda it: (it, 0, 0)),
            pl.BlockSpec((FE, TB * B), lambda it: (0, it)),
            pl.BlockSpec((N_NUM, 1), lambda it: (0, 0)),
            pl.BlockSpec((FE, 1), lambda it: (0, 0)),
            pl.BlockSpec((N_NUM, 1), lambda it: (0, 0)),
            pl.BlockSpec((FE, 1), lambda it: (0, 0)),
        ],
        out_specs=pl.BlockSpec((TB, F, B), lambda it: (it, 0, 0)),
        out_shape=jax.ShapeDtypeStruct((T, F, B), jnp.float32),
        compiler_params=pltpu.CompilerParams(
            vmem_limit_bytes=100 * 1024 * 1024),
    )(len_row, numeric_t, emb_fm, s_n, s_e, c_n, c_e)
    return jnp.transpose(out_t, (2, 0, 1))


def kernel(numeric, emb_idx, lengths, tables, gamma, beta):
    tables_2d = jnp.transpose(tables, (0, 2, 1)).reshape(FE, VOCAB)
    idx_tm = jnp.transpose(emb_idx, (2, 1, 0)).reshape(N_EMB, BT)
    idx_tm = idx_tm.astype(jnp.int32)
    numeric_t = jnp.transpose(numeric, (1, 2, 0))
    len_row = lengths.reshape(1, B).astype(jnp.int32)
    mask_tm = (jnp.arange(T, dtype=jnp.int32)[:, None]
               < lengths[None, :]).astype(jnp.float32).reshape(1, BT)
    emb_fm = _sc_gather(tables_2d, idx_tm)
    gamma_n = gamma[:N_NUM].reshape(N_NUM, 1)
    gamma_e = gamma[N_NUM:].reshape(FE, 1)
    beta_n = beta[:N_NUM].reshape(N_NUM, 1)
    beta_e = beta[N_NUM:].reshape(FE, 1)
    s_n, s_e, c_n, c_e = _stats(len_row, numeric_t, mask_tm, emb_fm,
                                gamma_n, gamma_e, beta_n, beta_e)
    return _norm(len_row, numeric_t, emb_fm, s_n, s_e, c_n, c_e)


# speedup vs baseline: 1.1788x; 1.0031x over previous
import functools

import jax
import jax.numpy as jnp
from jax import lax
from jax.experimental import pallas as pl
from jax.experimental.pallas import tpu as pltpu
from jax.experimental.pallas import tpu_sc as plsc

B, T = 1024, 50
N_NUM, N_EMB = 13, 26
EMB_DIM = 32
VOCAB = 100000
F = N_NUM + N_EMB * EMB_DIM
FE = N_EMB * EMB_DIM
EPS = 1e-5
BT = B * T

NC, NS = 2, 16
NW = NC * NS
RPW = FE // NW
TCK = 6400
NTC = BT // TCK


def _sc_gather(tables_2d, idx_tm):
    mesh = plsc.VectorSubcoreMesh(core_axis_name="c", subcore_axis_name="s")

    @functools.partial(
        pl.kernel,
        mesh=mesh,
        out_type=jax.ShapeDtypeStruct((FE, BT), jnp.float32),
        scratch_types=[
            pltpu.VMEM((VOCAB,), jnp.float32),
            pltpu.VMEM((2, TCK), jnp.int32),
            pltpu.VMEM((2, TCK), jnp.float32),
            pltpu.SemaphoreType.DMA,
            pltpu.SemaphoreType.DMA,
            pltpu.SemaphoreType.DMA,
            pltpu.SemaphoreType.DMA,
        ],
        compiler_params=pltpu.CompilerParams(use_tc_tiling_on_sc=False,
                                             needs_layout_passes=False),
    )
    def gather_k(tbl_hbm, idx_hbm, out_hbm, row_v, idx_v, out_v,
                 isem_a, isem_b, osem_a, osem_b):
        wid = lax.axis_index("s") * NC + lax.axis_index("c")
        isems = (isem_a, isem_b)
        osems = (osem_a, osem_b)

        def idx_start(j, c, b):
            pltpu.async_copy(idx_hbm.at[j, pl.ds(c * TCK, TCK)],
                             idx_v.at[b], isems[b])

        def idx_wait(j, c, b):
            pltpu.make_async_copy(idx_hbm.at[j, pl.ds(c * TCK, TCK)],
                                  idx_v.at[b], isems[b]).wait()

        def out_start(rf, c, b):
            pltpu.async_copy(out_v.at[b],
                             out_hbm.at[rf, pl.ds(c * TCK, TCK)], osems[b])

        def out_wait(rf, c, b):
            pltpu.make_async_copy(
                out_v.at[b], out_hbm.at[rf, pl.ds(c * TCK, TCK)],
                osems[b]).wait()

        def compute(b):
            def gbody(g, _):
                base = g * 256
                for u in range(16):
                    iv = idx_v[b, pl.ds(base + u * 16, 16)]
                    out_v[b, pl.ds(base + u * 16, 16)] = (
                        plsc.load_gather(row_v, [iv]))
                return 0

            lax.fori_loop(0, TCK // 256, gbody, 0)

        def rbody(rr, _):
            rf = wid * RPW + rr
            j = rf // EMB_DIM
            pltpu.sync_copy(tbl_hbm.at[rf], row_v)
            idx_start(j, 0, 0)
            idx_start(j, 1, 1)

            def pbody(p, _):
                for b in range(2):
                    c = p * 2 + b
                    idx_wait(j, c, b)

                    @pl.when(c >= 2)
                    def _():
                        out_wait(rf, c - 2, b)

                    compute(b)

                    @pl.when(c + 2 < NTC)
                    def _():
                        idx_start(j, c + 2, b)

                    out_start(rf, c, b)
                return 0

            lax.fori_loop(0, NTC // 2, pbody, 0)
            out_wait(rf, NTC - 2, 0)
            out_wait(rf, NTC - 1, 1)
            return 0

        lax.fori_loop(0, RPW, rbody, 0)

    return gather_k(tables_2d, idx_tm)


CK = 2048
NBC = BT // CK


def _stats_body(len_ref, num_ref, mask_ref, emb_ref,
                gn_ref, ge_ref, bn_ref, be_ref,
                sn_ref, se_ref, cn_ref, ce_ref,
                acc_sn, acc_qn, acc_se, acc_qe, acc_n):
    i = pl.program_id(0)

    @pl.when(i == 0)
    def _():
        lens = len_ref[...]
        lf = lens.astype(jnp.float32)
        acc_n[...] = jnp.sum(lf).reshape(1, 1)
        m3 = (lax.broadcasted_iota(jnp.int32, (T, 1, B), 0)
              < lens.reshape(1, 1, B)).astype(jnp.float32)
        num = num_ref[...]
        nm = num * m3
        acc_sn[...] = jnp.sum(nm, axis=(0, 2)).reshape(N_NUM, 1)
        acc_qn[...] = jnp.sum(nm * num, axis=(0, 2)).reshape(N_NUM, 1)
        acc_se[...] = jnp.zeros_like(acc_se)
        acc_qe[...] = jnp.zeros_like(acc_qe)

    emb = emb_ref[...]
    em = emb * mask_ref[...]
    acc_se[...] += jnp.sum(em, axis=1).reshape(FE, 1)
    acc_qe[...] += jnp.sum(em * emb, axis=1).reshape(FE, 1)

    @pl.when(i == pl.num_programs(0) - 1)
    def _():
        inv_n = 1.0 / acc_n[0, 0]
        mean_n = acc_sn[...] * inv_n
        var_n = jnp.maximum(acc_qn[...] * inv_n - mean_n * mean_n, 0.0)
        s_n = gn_ref[...] * lax.rsqrt(var_n + EPS)
        sn_ref[...] = s_n
        cn_ref[...] = bn_ref[...] - mean_n * s_n
        mean_e = acc_se[...] * inv_n
        var_e = jnp.maximum(acc_qe[...] * inv_n - mean_e * mean_e, 0.0)
        s_e = ge_ref[...] * lax.rsqrt(var_e + EPS)
        se_ref[...] = s_e
        ce_ref[...] = be_ref[...] - mean_e * s_e


def _stats(len_row, numeric_t, mask_tm, emb_fm,
           gamma_n, gamma_e, beta_n, beta_e):
    return pl.pallas_call(
        _stats_body,
        grid=(NBC,),
        in_specs=[
            pl.BlockSpec((1, B), lambda i: (0, 0)),
            pl.BlockSpec((T, N_NUM, B), lambda i: (0, 0, 0)),
            pl.BlockSpec((1, CK), lambda i: (0, i)),
            pl.BlockSpec((FE, CK), lambda i: (0, i)),
            pl.BlockSpec((N_NUM, 1), lambda i: (0, 0)),
            pl.BlockSpec((FE, 1), lambda i: (0, 0)),
            pl.BlockSpec((N_NUM, 1), lambda i: (0, 0)),
            pl.BlockSpec((FE, 1), lambda i: (0, 0)),
        ],
        out_specs=[
            pl.BlockSpec((N_NUM, 1), lambda i: (0, 0)),
            pl.BlockSpec((FE, 1), lambda i: (0, 0)),
            pl.BlockSpec((N_NUM, 1), lambda i: (0, 0)),
            pl.BlockSpec((FE, 1), lambda i: (0, 0)),
        ],
        out_shape=[
            jax.ShapeDtypeStruct((N_NUM, 1), jnp.float32),
            jax.ShapeDtypeStruct((FE, 1), jnp.float32),
            jax.ShapeDtypeStruct((N_NUM, 1), jnp.float32),
            jax.ShapeDtypeStruct((FE, 1), jnp.float32),
        ],
        scratch_shapes=[
            pltpu.VMEM((N_NUM, 1), jnp.float32),
            pltpu.VMEM((N_NUM, 1), jnp.float32),
            pltpu.VMEM((FE, 1), jnp.float32),
            pltpu.VMEM((FE, 1), jnp.float32),
            pltpu.VMEM((1, 1), jnp.float32),
        ],
    )(len_row, numeric_t, mask_tm, emb_fm, gamma_n, gamma_e, beta_n, beta_e)


TB = 1


def _norm_body(len_ref, num_ref, emb_ref, sn_ref, se_ref, cn_ref, ce_ref,
               out_ref):
    it = pl.program_id(0)
    lens = len_ref[...]
    s_n, c_n = sn_ref[...], cn_ref[...]
    s_e, c_e = se_ref[...], ce_ref[...]
    for k in range(TB):
        t = it * TB + k
        m = (t < lens).astype(jnp.float32)
        e = emb_ref[:, k * B:(k + 1) * B]
        out_ref[k, N_NUM:F, :] = (e * s_e + c_e) * m
        nt = num_ref[k]
        out_ref[k, 0:N_NUM, :] = (nt * s_n + c_n) * m


def _norm(len_row, numeric_t, emb_fm, s_n, s_e, c_n, c_e):
    out_t = pl.pallas_call(
        _norm_body,
        grid=(T // TB,),
        in_specs=[
            pl.BlockSpec((1, B), lambda it: (0, 0)),
            pl.BlockSpec((TB, N_NUM, B), lambda it: (it, 0, 0)),
            pl.BlockSpec((FE, TB * B), lambda it: (0, it)),
            pl.BlockSpec((N_NUM, 1), lambda it: (0, 0)),
            pl.BlockSpec((FE, 1), lambda it: (0, 0)),
            pl.BlockSpec((N_NUM, 1), lambda it: (0, 0)),
            pl.BlockSpec((FE, 1), lambda it: (0, 0)),
        ],
        out_specs=pl.BlockSpec((TB, F, B), lambda it: (it, 0, 0)),
        out_shape=jax.ShapeDtypeStruct((T, F, B), jnp.float32),
        compiler_params=pltpu.CompilerParams(
            vmem_limit_bytes=100 * 1024 * 1024),
    )(len_row, numeric_t, emb_fm, s_n, s_e, c_n, c_e)
    return jnp.transpose(out_t, (2, 0, 1))


def kernel(numeric, emb_idx, lengths, tables, gamma, beta):
    tables_2d = jnp.transpose(tables, (0, 2, 1)).reshape(FE, VOCAB)
    idx_tm = jnp.transpose(emb_idx, (2, 1, 0)).reshape(N_EMB, BT)
    idx_tm = idx_tm.astype(jnp.int32)
    numeric_t = jnp.transpose(numeric, (1, 2, 0))
    len_row = lengths.reshape(1, B).astype(jnp.int32)
    mask_tm = (jnp.arange(T, dtype=jnp.int32)[:, None]
               < lengths[None, :]).astype(jnp.float32).reshape(1, BT)
    emb_fm = _sc_gather(tables_2d, idx_tm)
    gamma_n = gamma[:N_NUM].reshape(N_NUM, 1)
    gamma_e = gamma[N_NUM:].reshape(FE, 1)
    beta_n = beta[:N_NUM].reshape(N_NUM, 1)
    beta_e = beta[N_NUM:].reshape(FE, 1)
    s_n, s_e, c_n, c_e = _stats(len_row, numeric_t, mask_tm, emb_fm,
                                gamma_n, gamma_e, beta_n, beta_e)
    return _norm(len_row, numeric_t, emb_fm, s_n, s_e, c_n, c_e)
